# baseline (speedup 1.0000x reference)
import functools

import jax
import jax.numpy as jnp
from jax.experimental import pallas as pl

N = 2048
IN_CH = 128
OUT_CH = 128

_PREC = jax.lax.Precision.HIGHEST


def _dot(a, b):
    return jnp.dot(a, b, precision=_PREC, preferred_element_type=jnp.float32)


def _split_bf16(a):
    hi = a.astype(jnp.bfloat16)
    lo = (a - hi.astype(jnp.float32)).astype(jnp.bfloat16)
    return hi, lo


def _bdot(a, b):
    return jnp.dot(a, b, preferred_element_type=jnp.float32)


def _row_min_and_argmin(cur, iota):
    m = jnp.min(cur, axis=1, keepdims=True)
    jmin = jnp.min(jnp.where(cur == m, iota, cur.shape[1]), axis=1, keepdims=True)
    return m, jmin


def _d2_matrix(pos_dst, pos_src_t):
    dx = pos_dst[:, 0:1] - pos_src_t[0:1, :]
    dy = pos_dst[:, 1:2] - pos_src_t[1:2, :]
    dz = pos_dst[:, 2:3] - pos_src_t[2:3, :]
    return (dx * dx + dy * dy) + dz * dz


def _fps_kernel(px_ref, py_ref, pz_ref,
                s1x_ref, s1y_ref, s1z_ref,
                s2x_ref, s2y_ref, s2z_ref,
                s3x_ref, s3y_ref, s3z_ref):
    bsz = px_ref.shape[0]

    def run_level(px, py, pz, n_samples):
        n = px.shape[1]
        iota = jax.lax.broadcasted_iota(jnp.int32, (bsz, n), 1)

        def body(t, state):
            dists, far, sx, sy, sz = state
            ohf = iota == far
            pxf = jnp.sum(jnp.where(ohf, px, 0.0), axis=1, keepdims=True)
            pyf = jnp.sum(jnp.where(ohf, py, 0.0), axis=1, keepdims=True)
            pzf = jnp.sum(jnp.where(ohf, pz, 0.0), axis=1, keepdims=True)
            rec = jax.lax.broadcasted_iota(jnp.int32, (bsz, n_samples), 1) == t
            sx = jnp.where(rec, pxf, sx)
            sy = jnp.where(rec, pyf, sy)
            sz = jnp.where(rec, pzf, sz)
            dxx = px - pxf
            dyy = py - pyf
            dzz = pz - pzf
            d = (dxx * dxx + dyy * dyy) + dzz * dzz
            dists = jnp.minimum(dists, d)
            m = jnp.max(dists, axis=1, keepdims=True)
            far = jnp.min(jnp.where(dists == m, iota, n), axis=1, keepdims=True)
            return (dists, far, sx, sy, sz)

        state = (jnp.full((bsz, n), 1e10, dtype=jnp.float32),
                 jnp.zeros((bsz, 1), dtype=jnp.int32),
                 jnp.zeros((bsz, n_samples), dtype=jnp.float32),
                 jnp.zeros((bsz, n_samples), dtype=jnp.float32),
                 jnp.zeros((bsz, n_samples), dtype=jnp.float32))
        _, _, sx, sy, sz = jax.lax.fori_loop(0, n_samples, body, state)
        return sx, sy, sz

    s1x, s1y, s1z = run_level(px_ref[...], py_ref[...], pz_ref[...], s1x_ref.shape[1])
    s1x_ref[...], s1y_ref[...], s1z_ref[...] = s1x, s1y, s1z
    s2x, s2y, s2z = run_level(s1x, s1y, s1z, s2x_ref.shape[1])
    s2x_ref[...], s2y_ref[...], s2z_ref[...] = s2x, s2y, s2z
    s3x, s3y, s3z = run_level(s2x, s2y, s2z, s3x_ref.shape[1])
    s3x_ref[...], s3y_ref[...], s3z_ref[...] = s3x, s3y, s3z


def _run_fps(pos_b, bsz):
    px = pos_b[:, :, 0]
    py = pos_b[:, :, 1]
    pz = pos_b[:, :, 2]
    s1, s2, s3 = N // 2, N // 8, (N // 8) // 10
    f32 = jnp.float32
    outs = pl.pallas_call(
        _fps_kernel,
        out_shape=tuple(jax.ShapeDtypeStruct((bsz, s), f32)
                        for s in (s1, s1, s1, s2, s2, s2, s3, s3, s3)),
    )(px, py, pz)
    return outs[0:3], outs[3:6], outs[6:9]


def _sa_kernel(pos_src_t_ref, pos_src_ref, pos_dst_ref, x_ref,
               w1x_ref, w1p_ref, b1_ref, w2_ref, b2_ref, w3_ref, b3_ref,
               out_ref, *, r2, k):
    pos_src_t = pos_src_t_ref[0]
    pos_src = pos_src_ref[0]
    pos_dst = pos_dst_ref[0]
    x = x_ref[0]
    w1p = w1p_ref[...]

    d2 = _d2_matrix(pos_dst, pos_src_t)
    s, n = d2.shape

    u = _dot(x, w1x_ref[...])
    u = u + (pos_src[:, 0:1] * w1p[0:1, :]
             + pos_src[:, 1:2] * w1p[1:2, :]
             + pos_src[:, 2:3] * w1p[2:3, :])
    v = (pos_dst[:, 0:1] * w1p[0:1, :]
         + pos_dst[:, 1:2] * w1p[1:2, :]
         + pos_dst[:, 2:3] * w1p[2:3, :])

    u_hi, u_lo = _split_bf16(u)
    uhl = jnp.concatenate([u_hi, u_lo], axis=1)
    h1w = u.shape[1]

    iota = jax.lax.broadcasted_iota(jnp.int32, (s, n), 1)
    b1 = b1_ref[...]
    b2 = b2_ref[...]
    b3 = b3_ref[...]
    w2_hi, w2_lo = _split_bf16(w2_ref[...])
    w2hl = jnp.concatenate([w2_hi, w2_lo], axis=1)
    w3_hi, w3_lo = _split_bf16(w3_ref[...])
    w3hl = jnp.concatenate([w3_hi, w3_lo], axis=1)
    h2w = w2_hi.shape[1]
    co = w3_hi.shape[1]

    mask_any = jnp.min(d2, axis=1, keepdims=True) < r2

    def mlp_max(ohb, m, outmax):
        g = _bdot(ohb, uhl)
        uk = g[:, :h1w] + g[:, h1w:]
        h1 = jax.nn.relu(uk - v + b1)
        h1_hi, h1_lo = _split_bf16(h1)
        g2 = _bdot(h1_hi, w2hl)
        h2 = jax.nn.relu(g2[:, :h2w] + g2[:, h2w:] + _bdot(h1_lo, w2_hi) + b2)
        h2_hi, h2_lo = _split_bf16(h2)
        g3 = _bdot(h2_hi, w3hl)
        msg = g3[:, :co] + g3[:, co:] + _bdot(h2_lo, w3_hi) + b3
        msg = jnp.where(m < r2, msg, -jnp.inf)
        return jnp.maximum(outmax, msg)

    def body(_, state):
        cur, ohb_prev, m_prev, outmax = state
        outmax = mlp_max(ohb_prev, m_prev, outmax)
        m, jmin = _row_min_and_argmin(cur, iota)
        oh = iota == jmin
        cur = jnp.where(oh, jnp.inf, cur)
        return (cur, oh.astype(jnp.bfloat16), m, outmax)

    init = (d2, jnp.zeros((s, n), dtype=jnp.bfloat16),
            jnp.full((s, 1), jnp.inf, dtype=jnp.float32),
            jnp.full((s, co), -jnp.inf, dtype=jnp.float32))
    _, ohb_last, m_last, outmax = jax.lax.fori_loop(0, k, body, init)
    outmax = mlp_max(ohb_last, m_last, outmax)
    out_ref[0] = jnp.where(mask_any, outmax, 0.0)


def _run_sa(x_b, pos_src_xyz, pos_dst_xyz, params, r, bsz):
    (w1, b1), (w2, b2), (w3, b3) = params
    psx, psy, psz = pos_src_xyz
    pdx, pdy, pdz = pos_dst_xyz
    n = psx.shape[1]
    s = pdx.shape[1]
    ci = x_b.shape[2]
    h1 = w1.shape[1]
    h2 = w2.shape[1]
    co = w3.shape[1]
    pos_src_t = jnp.stack([psx, psy, psz], axis=1)
    pos_src = jnp.stack([psx, psy, psz], axis=2)
    pos_dst = jnp.stack([pdx, pdy, pdz], axis=2)
    w1x = w1[:ci]
    w1p = w1[ci:]

    fixed = lambda *shape: pl.BlockSpec(shape, lambda b: (0,) * len(shape))
    out = pl.pallas_call(
        functools.partial(_sa_kernel, r2=r * r, k=min(64, n)),
        grid=(bsz,),
        in_specs=[
            pl.BlockSpec((1, 3, n), lambda b: (b, 0, 0)),
            pl.BlockSpec((1, n, 3), lambda b: (b, 0, 0)),
            pl.BlockSpec((1, s, 3), lambda b: (b, 0, 0)),
            pl.BlockSpec((1, n, ci), lambda b: (b, 0, 0)),
            fixed(ci, h1), fixed(3, h1), fixed(1, h1), fixed(h1, h2), fixed(1, h2),
            fixed(h2, co), fixed(1, co),
        ],
        out_specs=pl.BlockSpec((1, s, co), lambda b: (b, 0, 0)),
        out_shape=jax.ShapeDtypeStruct((bsz, s, co), jnp.float32),
    )(pos_src_t, pos_src, pos_dst, x_b,
      w1x, w1p, b1.reshape(1, h1), w2, b2.reshape(1, h2), w3, b3.reshape(1, co))
    return out


def _sa_batched_kernel(pos_src_t_ref, pos_src_ref, pos_dst_ref, x_ref,
                       w1x_ref, w1p_ref, b1_ref, w2_ref, b2_ref, w3_ref, b3_ref,
                       out_ref, gmax_ref, *, r2, k, s_valid):
    bsz, _, n = pos_src_t_ref.shape
    sp = pos_dst_ref.shape[1]
    w1p = w1p_ref[...]

    d2_list, uhl_list, v_list = [], [], []
    for b in range(bsz):
        pos_src_t = pos_src_t_ref[b]
        pos_src = pos_src_ref[b]
        pos_dst = pos_dst_ref[b]
        d2_list.append(_d2_matrix(pos_dst, pos_src_t))
        u = _dot(x_ref[b], w1x_ref[...])
        u = u + (pos_src[:, 0:1] * w1p[0:1, :]
                 + pos_src[:, 1:2] * w1p[1:2, :]
                 + pos_src[:, 2:3] * w1p[2:3, :])
        u_hi, u_lo = _split_bf16(u)
        uhl_list.append(jnp.concatenate([u_hi, u_lo], axis=1))
        v_list.append(pos_dst[:, 0:1] * w1p[0:1, :]
                      + pos_dst[:, 1:2] * w1p[1:2, :]
                      + pos_dst[:, 2:3] * w1p[2:3, :])
    dd = jnp.concatenate(d2_list, axis=0)
    vv = jnp.concatenate(v_list, axis=0)
    h1w = vv.shape[1]
    rows = bsz * sp

    iota = jax.lax.broadcasted_iota(jnp.int32, (rows, n), 1)
    b1 = b1_ref[...]
    b2 = b2_ref[...]
    b3 = b3_ref[...]
    w2_hi, w2_lo = _split_bf16(w2_ref[...])
    w2hl = jnp.concatenate([w2_hi, w2_lo], axis=1)
    w3_hi, w3_lo = _split_bf16(w3_ref[...])
    w3hl = jnp.concatenate([w3_hi, w3_lo], axis=1)
    h2w = w2_hi.shape[1]
    co = w3_hi.shape[1]

    mask_any = jnp.min(dd, axis=1, keepdims=True) < r2

    def body(_, state):
        cur, outmax = state
        m, jmin = _row_min_and_argmin(cur, iota)
        oh = iota == jmin
        cur = jnp.where(oh, jnp.inf, cur)
        ohb = oh.astype(jnp.bfloat16)
        g = jnp.concatenate(
            [_bdot(ohb[b * sp:(b + 1) * sp], uhl_list[b]) for b in range(bsz)],
            axis=0)
        uk = g[:, :h1w] + g[:, h1w:]
        h1 = jax.nn.relu(uk - vv + b1)
        h1_hi, h1_lo = _split_bf16(h1)
        g2 = _bdot(h1_hi, w2hl)
        h2 = jax.nn.relu(g2[:, :h2w] + g2[:, h2w:] + _bdot(h1_lo, w2_hi) + b2)
        h2_hi, h2_lo = _split_bf16(h2)
        g3 = _bdot(h2_hi, w3hl)
        msg = g3[:, :co] + g3[:, co:] + _bdot(h2_lo, w3_hi) + b3
        msg = jnp.where(m < r2, msg, -jnp.inf)
        outmax = jnp.maximum(outmax, msg)
        return (cur, outmax)

    init = (dd, jnp.full((rows, co), -jnp.inf, dtype=jnp.float32))
    _, outmax = jax.lax.fori_loop(0, k, body, init)
    out = jnp.where(mask_any, outmax, 0.0)
    out_ref[...] = out.reshape(bsz, sp, co)
    gmax_ref[...] = jnp.concatenate(
        [jnp.max(out[b * sp:b * sp + s_valid], axis=0, keepdims=True)
         for b in range(bsz)], axis=0)


def _run_sa_batched(x_b, pos_src_xyz, pos_dst_xyz, params, r, bsz):
    (w1, b1), (w2, b2), (w3, b3) = params
    psx, psy, psz = pos_src_xyz
    pdx, pdy, pdz = pos_dst_xyz
    n = psx.shape[1]
    s = pdx.shape[1]
    sp = (s + 7) // 8 * 8
    if sp != s:
        pad = ((0, 0), (0, sp - s))
        pdx = jnp.pad(pdx, pad)
        pdy = jnp.pad(pdy, pad)
        pdz = jnp.pad(pdz, pad)
    ci = x_b.shape[2]
    h1 = w1.shape[1]
    h2 = w2.shape[1]
    co = w3.shape[1]
    pos_src_t = jnp.stack([psx, psy, psz], axis=1)
    pos_src = jnp.stack([psx, psy, psz], axis=2)
    pos_dst = jnp.stack([pdx, pdy, pdz], axis=2)

    out, gmax = pl.pallas_call(
        functools.partial(_sa_batched_kernel, r2=r * r, k=min(64, n), s_valid=s),
        out_shape=(jax.ShapeDtypeStruct((bsz, sp, co), jnp.float32),
                   jax.ShapeDtypeStruct((bsz, co), jnp.float32)),
    )(pos_src_t, pos_src, pos_dst, x_b,
      w1[:ci], w1[ci:], b1.reshape(1, h1), w2, b2.reshape(1, h2),
      w3, b3.reshape(1, co))
    return out[:, :s], gmax


def _fp_kernel(pos_src_t_ref, pos_dst_ref, xsrc_ref, xskip_ref,
               w1a_ref, w1b_ref, b1_ref, w2_ref, b2_ref, out_ref, *, kk):
    pos_src_t = pos_src_t_ref[0]
    pos_dst = pos_dst_ref[0]
    d2 = _d2_matrix(pos_dst, pos_src_t)
    nd, ns = d2.shape
    iota = jax.lax.broadcasted_iota(jnp.int32, (nd, ns), 1)

    cur = d2
    amat = jnp.zeros((nd, ns), dtype=jnp.float32)
    sumw = jnp.zeros((nd, 1), dtype=jnp.float32)
    for _ in range(kk):
        m, jmin = _row_min_and_argmin(cur, iota)
        oh = iota == jmin
        cur = jnp.where(oh, jnp.inf, cur)
        w = 1.0 / (jnp.sqrt(m + 1e-12) + 1e-08)
        amat = amat + jnp.where(oh, w, 0.0)
        sumw = sumw + w

    interp = _dot(amat, xsrc_ref[0]) / (sumw + 1e-08)
    h = jax.nn.relu(_dot(interp, w1a_ref[...])
                    + _dot(xskip_ref[0], w1b_ref[...]) + b1_ref[...])
    out_ref[0] = _dot(h, w2_ref[...]) + b2_ref[...]


def _run_fp(xsrc_b, pos_src_xyz, xskip_b, pos_dst_xyz, params, bsz):
    (w1, b1), (w2, b2) = params
    psx, psy, psz = pos_src_xyz
    pdx, pdy, pdz = pos_dst_xyz
    ns = psx.shape[1]
    nd = pdx.shape[1]
    c = xsrc_b.shape[2]
    cs = xskip_b.shape[2]
    h1 = w1.shape[1]
    co = w2.shape[1]
    pos_src_t = jnp.stack([psx, psy, psz], axis=1)
    pos_dst = jnp.stack([pdx, pdy, pdz], axis=2)
    w1a = w1[:c]
    w1b = w1[c:]

    fixed = lambda *shape: pl.BlockSpec(shape, lambda b: (0,) * len(shape))
    out = pl.pallas_call(
        functools.partial(_fp_kernel, kk=min(3, ns)),
        grid=(bsz,),
        in_specs=[
            pl.BlockSpec((1, 3, ns), lambda b: (b, 0, 0)),
            pl.BlockSpec((1, nd, 3), lambda b: (b, 0, 0)),
            pl.BlockSpec((1, ns, c), lambda b: (b, 0, 0)),
            pl.BlockSpec((1, nd, cs), lambda b: (b, 0, 0)),
            fixed(c, h1), fixed(cs, h1), fixed(1, h1), fixed(h1, co), fixed(1, co),
        ],
        out_specs=pl.BlockSpec((1, nd, co), lambda b: (b, 0, 0)),
        out_shape=jax.ShapeDtypeStruct((bsz, nd, co), jnp.float32),
    )(pos_src_t, pos_dst, xsrc_b, xskip_b,
      w1a, w1b, b1.reshape(1, h1), w2, b2.reshape(1, co))
    return out


def _gmax_kernel(x_ref, o_ref):
    o_ref[...] = jnp.max(x_ref[...], axis=1)


def _run_gmax(s3x_b, bsz):
    co = s3x_b.shape[2]
    return pl.pallas_call(
        _gmax_kernel,
        out_shape=jax.ShapeDtypeStruct((bsz, co), jnp.float32),
    )(s3x_b)


_ABLATE = 0


def kernel(x, pos, batch, sa1_params, sa2_params, sa3_params, fp3_params, fp2_params, fp1_params):
    bsz = x.shape[0] // N
    if _ABLATE:
        x_b = x.reshape(bsz, N, IN_CH)
        pos_b = pos.reshape(bsz, N, 3)
        p0 = (pos_b[:, :, 0], pos_b[:, :, 1], pos_b[:, :, 2])
        p1, p2, p3 = _run_fps(pos_b, bsz)
        acc = p1[0].sum() + p2[0].sum() + p3[0].sum()
        if _ABLATE >= 2:
            s1x = _run_sa(x_b, p0, p1, sa1_params, 10.0, bsz)
            acc = acc + s1x.sum()
        if _ABLATE >= 3:
            s2x = _run_sa(s1x, p1, p2, sa2_params, 20.0, bsz)
            acc = acc + s2x.sum()
        if _ABLATE >= 4:
            s3x = _run_sa(s2x, p2, p3, sa3_params, 40.0, bsz)
            acc = acc + _run_gmax(s3x, bsz).sum()
        f1 = jnp.zeros((bsz * N, OUT_CH), jnp.float32) + acc
        gfeat = jnp.zeros((bsz, 512), jnp.float32) + acc
        return f1, gfeat
    x_b = x.reshape(bsz, N, IN_CH)
    pos_b = pos.reshape(bsz, N, 3)

    p0 = (pos_b[:, :, 0], pos_b[:, :, 1], pos_b[:, :, 2])
    p1, p2, p3 = _run_fps(pos_b, bsz)

    s1x = _run_sa(x_b, p0, p1, sa1_params, 10.0, bsz)
    s2x, _ = _run_sa_batched(s1x, p1, p2, sa2_params, 20.0, bsz)
    s3x, gfeat = _run_sa_batched(s2x, p2, p3, sa3_params, 40.0, bsz)

    f3 = _run_fp(s3x, p3, s2x, p2, fp3_params, bsz)
    f2 = _run_fp(f3, p2, s1x, p1, fp2_params, bsz)
    f1 = _run_fp(f2, p1, x_b, p0, fp1_params, bsz)

    return f1.reshape(bsz * N, OUT_CH), gfeat


# speedup vs baseline: 1.1931x; 1.1931x over previous
import functools

import jax
import jax.numpy as jnp
from jax.experimental import pallas as pl

N = 2048
IN_CH = 128
OUT_CH = 128

_PREC = jax.lax.Precision.HIGHEST


def _dot(a, b):
    return jnp.dot(a, b, precision=_PREC, preferred_element_type=jnp.float32)


def _split_bf16(a):
    hi = a.astype(jnp.bfloat16)
    lo = (a - hi.astype(jnp.float32)).astype(jnp.bfloat16)
    return hi, lo


def _bdot(a, b):
    return jnp.dot(a, b, preferred_element_type=jnp.float32)


def _row_min_and_argmin(cur, iota):
    m = jnp.min(cur, axis=1, keepdims=True)
    jmin = jnp.min(jnp.where(cur == m, iota, cur.shape[1]), axis=1, keepdims=True)
    return m, jmin


def _d2_matrix(pos_dst, pos_src_t):
    dx = pos_dst[:, 0:1] - pos_src_t[0:1, :]
    dy = pos_dst[:, 1:2] - pos_src_t[1:2, :]
    dz = pos_dst[:, 2:3] - pos_src_t[2:3, :]
    return (dx * dx + dy * dy) + dz * dz


def _fps_kernel(px_ref, py_ref, pz_ref,
                s1x_ref, s1y_ref, s1z_ref,
                s2x_ref, s2y_ref, s2z_ref,
                s3x_ref, s3y_ref, s3z_ref):
    bsz = px_ref.shape[0]

    def run_level(px, py, pz, n_samples):
        n = px.shape[1]
        iota = jax.lax.broadcasted_iota(jnp.int32, (bsz, n), 1)

        def body(t, state):
            dists, far, sx, sy, sz = state
            ohf = iota == far
            pxf = jnp.sum(jnp.where(ohf, px, 0.0), axis=1, keepdims=True)
            pyf = jnp.sum(jnp.where(ohf, py, 0.0), axis=1, keepdims=True)
            pzf = jnp.sum(jnp.where(ohf, pz, 0.0), axis=1, keepdims=True)
            rec = jax.lax.broadcasted_iota(jnp.int32, (bsz, n_samples), 1) == t
            sx = jnp.where(rec, pxf, sx)
            sy = jnp.where(rec, pyf, sy)
            sz = jnp.where(rec, pzf, sz)
            dxx = px - pxf
            dyy = py - pyf
            dzz = pz - pzf
            d = (dxx * dxx + dyy * dyy) + dzz * dzz
            dists = jnp.minimum(dists, d)
            m = jnp.max(dists, axis=1, keepdims=True)
            far = jnp.min(jnp.where(dists == m, iota, n), axis=1, keepdims=True)
            return (dists, far, sx, sy, sz)

        state = (jnp.full((bsz, n), 1e10, dtype=jnp.float32),
                 jnp.zeros((bsz, 1), dtype=jnp.int32),
                 jnp.zeros((bsz, n_samples), dtype=jnp.float32),
                 jnp.zeros((bsz, n_samples), dtype=jnp.float32),
                 jnp.zeros((bsz, n_samples), dtype=jnp.float32))
        _, _, sx, sy, sz = jax.lax.fori_loop(0, n_samples, body, state)
        return sx, sy, sz

    s1x, s1y, s1z = run_level(px_ref[...], py_ref[...], pz_ref[...], s1x_ref.shape[1])
    s1x_ref[...], s1y_ref[...], s1z_ref[...] = s1x, s1y, s1z
    s2x, s2y, s2z = run_level(s1x, s1y, s1z, s2x_ref.shape[1])
    s2x_ref[...], s2y_ref[...], s2z_ref[...] = s2x, s2y, s2z
    s3x, s3y, s3z = run_level(s2x, s2y, s2z, s3x_ref.shape[1])
    s3x_ref[...], s3y_ref[...], s3z_ref[...] = s3x, s3y, s3z


def _run_fps(pos_b, bsz):
    px = pos_b[:, :, 0]
    py = pos_b[:, :, 1]
    pz = pos_b[:, :, 2]
    s1, s2, s3 = N // 2, N // 8, (N // 8) // 10
    f32 = jnp.float32
    outs = pl.pallas_call(
        _fps_kernel,
        out_shape=tuple(jax.ShapeDtypeStruct((bsz, s), f32)
                        for s in (s1, s1, s1, s2, s2, s2, s3, s3, s3)),
    )(px, py, pz)
    return outs[0:3], outs[3:6], outs[6:9]


def _sa_kernel(pos_src_t_ref, pos_src_ref, pos_dst_ref, x_ref,
               w1x_ref, w1p_ref, b1_ref, w2_ref, b2_ref, w3_ref, b3_ref,
               out_ref, *, r2, k):
    pos_src_t = pos_src_t_ref[0]
    pos_src = pos_src_ref[0]
    pos_dst = pos_dst_ref[0]
    x = x_ref[0]
    w1p = w1p_ref[...]

    d2 = _d2_matrix(pos_dst, pos_src_t)
    s, n = d2.shape

    u = _dot(x, w1x_ref[...])
    u = u + (pos_src[:, 0:1] * w1p[0:1, :]
             + pos_src[:, 1:2] * w1p[1:2, :]
             + pos_src[:, 2:3] * w1p[2:3, :])
    v = (pos_dst[:, 0:1] * w1p[0:1, :]
         + pos_dst[:, 1:2] * w1p[1:2, :]
         + pos_dst[:, 2:3] * w1p[2:3, :])

    u_hi, u_lo = _split_bf16(u)
    uhl = jnp.concatenate([u_hi, u_lo], axis=1)
    h1w = u.shape[1]

    iota = jax.lax.broadcasted_iota(jnp.int32, (s, n), 1)
    b1 = b1_ref[...]
    b2 = b2_ref[...]
    b3 = b3_ref[...]
    w2_hi, w2_lo = _split_bf16(w2_ref[...])
    w2hl = jnp.concatenate([w2_hi, w2_lo], axis=1)
    w3_hi, w3_lo = _split_bf16(w3_ref[...])
    w3hl = jnp.concatenate([w3_hi, w3_lo], axis=1)
    h2w = w2_hi.shape[1]
    co = w3_hi.shape[1]

    mask_any = jnp.min(d2, axis=1, keepdims=True) < r2

    def body(_, state):
        cur, outmax = state
        m, jmin = _row_min_and_argmin(cur, iota)
        oh = iota == jmin
        cur = jnp.where(oh, jnp.inf, cur)
        g = _bdot(oh.astype(jnp.bfloat16), uhl)
        uk = g[:, :h1w] + g[:, h1w:]
        h1 = jax.nn.relu(uk - v + b1)
        h1_hi, h1_lo = _split_bf16(h1)
        g2 = _bdot(h1_hi, w2hl)
        h2 = jax.nn.relu(g2[:, :h2w] + g2[:, h2w:] + _bdot(h1_lo, w2_hi) + b2)
        h2_hi, h2_lo = _split_bf16(h2)
        g3 = _bdot(h2_hi, w3hl)
        msg = g3[:, :co] + g3[:, co:] + _bdot(h2_lo, w3_hi) + b3
        msg = jnp.where(m < r2, msg, -jnp.inf)
        outmax = jnp.maximum(outmax, msg)
        return (cur, outmax)

    init = (d2, jnp.full((s, co), -jnp.inf, dtype=jnp.float32))
    _, outmax = jax.lax.fori_loop(0, k, body, init)
    out_ref[0] = jnp.where(mask_any, outmax, 0.0)


def _run_sa(x_b, pos_src_xyz, pos_dst_xyz, params, r, bsz):
    (w1, b1), (w2, b2), (w3, b3) = params
    psx, psy, psz = pos_src_xyz
    pdx, pdy, pdz = pos_dst_xyz
    n = psx.shape[1]
    s = pdx.shape[1]
    ci = x_b.shape[2]
    h1 = w1.shape[1]
    h2 = w2.shape[1]
    co = w3.shape[1]
    pos_src_t = jnp.stack([psx, psy, psz], axis=1)
    pos_src = jnp.stack([psx, psy, psz], axis=2)
    pos_dst = jnp.stack([pdx, pdy, pdz], axis=2)
    w1x = w1[:ci]
    w1p = w1[ci:]

    fixed = lambda *shape: pl.BlockSpec(shape, lambda b: (0,) * len(shape))
    out = pl.pallas_call(
        functools.partial(_sa_kernel, r2=r * r, k=min(64, n)),
        grid=(bsz,),
        in_specs=[
            pl.BlockSpec((1, 3, n), lambda b: (b, 0, 0)),
            pl.BlockSpec((1, n, 3), lambda b: (b, 0, 0)),
            pl.BlockSpec((1, s, 3), lambda b: (b, 0, 0)),
            pl.BlockSpec((1, n, ci), lambda b: (b, 0, 0)),
            fixed(ci, h1), fixed(3, h1), fixed(1, h1), fixed(h1, h2), fixed(1, h2),
            fixed(h2, co), fixed(1, co),
        ],
        out_specs=pl.BlockSpec((1, s, co), lambda b: (b, 0, 0)),
        out_shape=jax.ShapeDtypeStruct((bsz, s, co), jnp.float32),
    )(pos_src_t, pos_src, pos_dst, x_b,
      w1x, w1p, b1.reshape(1, h1), w2, b2.reshape(1, h2), w3, b3.reshape(1, co))
    return out


def _sa_batched_kernel(pos_src_t_ref, pos_src_ref, pos_dst_ref, x_ref,
                       w1x_ref, w1p_ref, b1_ref, w2_ref, b2_ref, w3_ref, b3_ref,
                       out_ref, gmax_ref, *, r2, k, s_valid):
    bsz, _, n = pos_src_t_ref.shape
    sp = pos_dst_ref.shape[1]
    w1p = w1p_ref[...]

    d2_list, uhl_list, v_list = [], [], []
    for b in range(bsz):
        pos_src_t = pos_src_t_ref[b]
        pos_src = pos_src_ref[b]
        pos_dst = pos_dst_ref[b]
        d2_list.append(_d2_matrix(pos_dst, pos_src_t))
        u = _dot(x_ref[b], w1x_ref[...])
        u = u + (pos_src[:, 0:1] * w1p[0:1, :]
                 + pos_src[:, 1:2] * w1p[1:2, :]
                 + pos_src[:, 2:3] * w1p[2:3, :])
        u_hi, u_lo = _split_bf16(u)
        uhl_list.append(jnp.concatenate([u_hi, u_lo], axis=1))
        v_list.append(pos_dst[:, 0:1] * w1p[0:1, :]
                      + pos_dst[:, 1:2] * w1p[1:2, :]
                      + pos_dst[:, 2:3] * w1p[2:3, :])
    dd = jnp.concatenate(d2_list, axis=0)
    vv = jnp.concatenate(v_list, axis=0)
    h1w = vv.shape[1]
    rows = bsz * sp

    iota = jax.lax.broadcasted_iota(jnp.int32, (rows, n), 1)
    b1 = b1_ref[...]
    b2 = b2_ref[...]
    b3 = b3_ref[...]
    w2_hi, w2_lo = _split_bf16(w2_ref[...])
    w2hl = jnp.concatenate([w2_hi, w2_lo], axis=1)
    w3_hi, w3_lo = _split_bf16(w3_ref[...])
    w3hl = jnp.concatenate([w3_hi, w3_lo], axis=1)
    h2w = w2_hi.shape[1]
    co = w3_hi.shape[1]

    mask_any = jnp.min(dd, axis=1, keepdims=True) < r2

    def body(_, state):
        cur, outmax = state
        m, jmin = _row_min_and_argmin(cur, iota)
        oh = iota == jmin
        cur = jnp.where(oh, jnp.inf, cur)
        ohb = oh.astype(jnp.bfloat16)
        g = jnp.concatenate(
            [_bdot(ohb[b * sp:(b + 1) * sp], uhl_list[b]) for b in range(bsz)],
            axis=0)
        uk = g[:, :h1w] + g[:, h1w:]
        h1 = jax.nn.relu(uk - vv + b1)
        h1_hi, h1_lo = _split_bf16(h1)
        g2 = _bdot(h1_hi, w2hl)
        h2 = jax.nn.relu(g2[:, :h2w] + g2[:, h2w:] + _bdot(h1_lo, w2_hi) + b2)
        h2_hi, h2_lo = _split_bf16(h2)
        g3 = _bdot(h2_hi, w3hl)
        msg = g3[:, :co] + g3[:, co:] + _bdot(h2_lo, w3_hi) + b3
        msg = jnp.where(m < r2, msg, -jnp.inf)
        outmax = jnp.maximum(outmax, msg)
        return (cur, outmax)

    init = (dd, jnp.full((rows, co), -jnp.inf, dtype=jnp.float32))
    _, outmax = jax.lax.fori_loop(0, k, body, init)
    out = jnp.where(mask_any, outmax, 0.0)
    out_ref[...] = out.reshape(bsz, sp, co)
    gmax_ref[...] = jnp.concatenate(
        [jnp.max(out[b * sp:b * sp + s_valid], axis=0, keepdims=True)
         for b in range(bsz)], axis=0)


def _run_sa_batched(x_b, pos_src_xyz, pos_dst_xyz, params, r, bsz):
    (w1, b1), (w2, b2), (w3, b3) = params
    psx, psy, psz = pos_src_xyz
    pdx, pdy, pdz = pos_dst_xyz
    n = psx.shape[1]
    s = pdx.shape[1]
    sp = (s + 7) // 8 * 8
    if sp != s:
        pad = ((0, 0), (0, sp - s))
        pdx = jnp.pad(pdx, pad)
        pdy = jnp.pad(pdy, pad)
        pdz = jnp.pad(pdz, pad)
    ci = x_b.shape[2]
    h1 = w1.shape[1]
    h2 = w2.shape[1]
    co = w3.shape[1]
    pos_src_t = jnp.stack([psx, psy, psz], axis=1)
    pos_src = jnp.stack([psx, psy, psz], axis=2)
    pos_dst = jnp.stack([pdx, pdy, pdz], axis=2)

    out, gmax = pl.pallas_call(
        functools.partial(_sa_batched_kernel, r2=r * r, k=min(64, n), s_valid=s),
        out_shape=(jax.ShapeDtypeStruct((bsz, sp, co), jnp.float32),
                   jax.ShapeDtypeStruct((bsz, co), jnp.float32)),
    )(pos_src_t, pos_src, pos_dst, x_b,
      w1[:ci], w1[ci:], b1.reshape(1, h1), w2, b2.reshape(1, h2),
      w3, b3.reshape(1, co))
    return out[:, :s], gmax


def _fp_kernel(pos_src_t_ref, pos_dst_ref, xsrc_ref, xskip_ref,
               w1a_ref, w1b_ref, b1_ref, w2_ref, b2_ref, out_ref, *, kk):
    pos_src_t = pos_src_t_ref[0]
    pos_dst = pos_dst_ref[0]
    d2 = _d2_matrix(pos_dst, pos_src_t)
    nd, ns = d2.shape
    iota = jax.lax.broadcasted_iota(jnp.int32, (nd, ns), 1)

    cur = d2
    amat = jnp.zeros((nd, ns), dtype=jnp.float32)
    sumw = jnp.zeros((nd, 1), dtype=jnp.float32)
    for _ in range(kk):
        m, jmin = _row_min_and_argmin(cur, iota)
        oh = iota == jmin
        cur = jnp.where(oh, jnp.inf, cur)
        w = 1.0 / (jnp.sqrt(m + 1e-12) + 1e-08)
        amat = amat + jnp.where(oh, w, 0.0)
        sumw = sumw + w

    interp = _dot(amat, xsrc_ref[0]) / (sumw + 1e-08)
    h = jax.nn.relu(_dot(interp, w1a_ref[...])
                    + _dot(xskip_ref[0], w1b_ref[...]) + b1_ref[...])
    out_ref[0] = _dot(h, w2_ref[...]) + b2_ref[...]


def _run_fp(xsrc_b, pos_src_xyz, xskip_b, pos_dst_xyz, params, bsz):
    (w1, b1), (w2, b2) = params
    psx, psy, psz = pos_src_xyz
    pdx, pdy, pdz = pos_dst_xyz
    ns = psx.shape[1]
    nd = pdx.shape[1]
    c = xsrc_b.shape[2]
    cs = xskip_b.shape[2]
    h1 = w1.shape[1]
    co = w2.shape[1]
    pos_src_t = jnp.stack([psx, psy, psz], axis=1)
    pos_dst = jnp.stack([pdx, pdy, pdz], axis=2)
    w1a = w1[:c]
    w1b = w1[c:]

    fixed = lambda *shape: pl.BlockSpec(shape, lambda b: (0,) * len(shape))
    out = pl.pallas_call(
        functools.partial(_fp_kernel, kk=min(3, ns)),
        grid=(bsz,),
        in_specs=[
            pl.BlockSpec((1, 3, ns), lambda b: (b, 0, 0)),
            pl.BlockSpec((1, nd, 3), lambda b: (b, 0, 0)),
            pl.BlockSpec((1, ns, c), lambda b: (b, 0, 0)),
            pl.BlockSpec((1, nd, cs), lambda b: (b, 0, 0)),
            fixed(c, h1), fixed(cs, h1), fixed(1, h1), fixed(h1, co), fixed(1, co),
        ],
        out_specs=pl.BlockSpec((1, nd, co), lambda b: (b, 0, 0)),
        out_shape=jax.ShapeDtypeStruct((bsz, nd, co), jnp.float32),
    )(pos_src_t, pos_dst, xsrc_b, xskip_b,
      w1a, w1b, b1.reshape(1, h1), w2, b2.reshape(1, co))
    return out


def kernel(x, pos, batch, sa1_params, sa2_params, sa3_params, fp3_params, fp2_params, fp1_params):
    bsz = x.shape[0] // N
    x_b = x.reshape(bsz, N, IN_CH)
    pos_b = pos.reshape(bsz, N, 3)

    p0 = (pos_b[:, :, 0], pos_b[:, :, 1], pos_b[:, :, 2])
    p1, p2, p3 = _run_fps(pos_b, bsz)

    s1x = _run_sa(x_b, p0, p1, sa1_params, 10.0, bsz)
    s2x, _ = _run_sa_batched(s1x, p1, p2, sa2_params, 20.0, bsz)
    s3x, gfeat = _run_sa_batched(s2x, p2, p3, sa3_params, 40.0, bsz)

    f3 = _run_fp(s3x, p3, s2x, p2, fp3_params, bsz)
    f2 = _run_fp(f3, p2, s1x, p1, fp2_params, bsz)
    f1 = _run_fp(f2, p1, x_b, p0, fp1_params, bsz)

    return f1.reshape(bsz * N, OUT_CH), gfeat
